# baseline (speedup 1.0000x reference)
import jax
import jax.numpy as jnp
from jax.experimental import pallas as pl
from jax.experimental.pallas import tpu as pltpu


def _pos_kernel(col_ref, row_ref, out_ref, scratch, sems):
    b = out_ref.shape[0]
    n = out_ref.shape[1]
    rows = col_ref.shape[0]
    d = col_ref.shape[1]
    coord = (jax.lax.broadcasted_iota(jnp.int32, (n, 1), 0).astype(jnp.float32)
             * (float(rows - 1) / n))
    fc = jnp.floor(coord)
    delta = coord - fc
    cols = jax.lax.broadcasted_iota(jnp.int32, (n, rows), 1).astype(jnp.float32)
    wmat = (jnp.where(cols == fc, 1.0 - delta, 0.0)
            + jnp.where(cols == fc + 1.0, delta, 0.0))
    xe = jnp.dot(wmat, col_ref[...], preferred_element_type=jnp.float32)
    ye = jnp.dot(wmat, row_ref[...], preferred_element_type=jnp.float32)
    xpart = jnp.broadcast_to(xe[None, :, :], (n, n, d))
    ypart = jnp.broadcast_to(ye[:, None, :], (n, n, d))
    scratch[...] = jnp.concatenate([xpart, ypart], axis=2)
    half = n // 2
    copies = [
        pltpu.make_async_copy(
            scratch.at[pl.ds(j * half, half)],
            out_ref.at[i, pl.ds(j * half, half)],
            sems.at[i, j],
        )
        for i in range(b)
        for j in range(2)
    ]
    for c in copies:
        c.start()
    for c in copies:
        c.wait()


def kernel(x, calibs, img_size, row_embed, col_embed):
    b = x.shape[0]
    h, w = x.shape[-2], x.shape[-1]
    d = row_embed.shape[1]
    out = pl.pallas_call(
        _pos_kernel,
        in_specs=[
            pl.BlockSpec(memory_space=pltpu.MemorySpace.VMEM),
            pl.BlockSpec(memory_space=pltpu.MemorySpace.VMEM),
        ],
        out_specs=pl.BlockSpec(memory_space=pltpu.MemorySpace.HBM),
        out_shape=jax.ShapeDtypeStruct((b, h, w, 2 * d), jnp.float32),
        scratch_shapes=[
            pltpu.VMEM((h, w, 2 * d), jnp.float32),
            pltpu.SemaphoreType.DMA((b, 2)),
        ],
    )(col_embed, row_embed)
    return jnp.transpose(out, (0, 3, 1, 2))


# speedup vs baseline: 1.0082x; 1.0082x over previous
import jax
import jax.numpy as jnp
from jax.experimental import pallas as pl
from jax.experimental.pallas import tpu as pltpu


def _pos_kernel(col_ref, row_ref, out_ref, scratch, sems):
    b = out_ref.shape[0]
    n = out_ref.shape[1]
    rows = col_ref.shape[0]
    d = col_ref.shape[1]
    coord = (jax.lax.broadcasted_iota(jnp.int32, (n, 1), 0).astype(jnp.float32)
             * (float(rows - 1) / n))
    fc = jnp.floor(coord)
    delta = coord - fc
    cols = jax.lax.broadcasted_iota(jnp.int32, (n, rows), 1).astype(jnp.float32)
    wmat = (jnp.where(cols == fc, 1.0 - delta, 0.0)
            + jnp.where(cols == fc + 1.0, delta, 0.0))
    xe = jnp.dot(wmat, col_ref[...], preferred_element_type=jnp.float32)
    ye = jnp.dot(wmat, row_ref[...], preferred_element_type=jnp.float32)
    xpart = jnp.broadcast_to(xe[None, :, :], (n, n, d))
    ypart = jnp.broadcast_to(ye[:, None, :], (n, n, d))
    scratch[...] = jnp.concatenate([xpart, ypart], axis=2)
    copies = [
        pltpu.make_async_copy(scratch, out_ref.at[i], sems.at[i])
        for i in range(b)
    ]
    for c in copies:
        c.start()
    for c in copies:
        c.wait()


def kernel(x, calibs, img_size, row_embed, col_embed):
    b = x.shape[0]
    h, w = x.shape[-2], x.shape[-1]
    d = row_embed.shape[1]
    out = pl.pallas_call(
        _pos_kernel,
        in_specs=[
            pl.BlockSpec(memory_space=pltpu.MemorySpace.VMEM),
            pl.BlockSpec(memory_space=pltpu.MemorySpace.VMEM),
        ],
        out_specs=pl.BlockSpec(memory_space=pltpu.MemorySpace.HBM),
        out_shape=jax.ShapeDtypeStruct((b, h, w, 2 * d), jnp.float32),
        scratch_shapes=[
            pltpu.VMEM((h, w, 2 * d), jnp.float32),
            pltpu.SemaphoreType.DMA((b,)),
        ],
    )(col_embed, row_embed)
    return jnp.transpose(out, (0, 3, 1, 2))
